# Initial kernel scaffold; baseline (speedup 1.0000x reference)
import jax
import jax.numpy as jnp
from jax.experimental import pallas as pl


def kernel(boxes, scores):
    raise NotImplementedError("write your pallas kernel here")


# speedup vs baseline: 24.3620x; 24.3620x over previous
import jax
import jax.numpy as jnp
from jax.experimental import pallas as pl

_N = 5000
_P = 5120
_ROWS = 40
_MAX_OUT = 300
_IOU_THRESH = 0.7
_NEG = -1e30


def _allmax(x):
    r = jnp.max(x, axis=1, keepdims=True)
    r = jnp.max(r, axis=0, keepdims=True)
    return jnp.broadcast_to(r, x.shape)


def _allmin(x):
    r = jnp.min(x, axis=1, keepdims=True)
    r = jnp.min(r, axis=0, keepdims=True)
    return jnp.broadcast_to(r, x.shape)


def _allsum(x):
    r = jnp.sum(x, axis=1, keepdims=True)
    r = jnp.sum(r, axis=0, keepdims=True)
    return jnp.broadcast_to(r, x.shape)


def _nms_body(x1_ref, y1_ref, x2_ref, y2_ref, sc_ref, out_ref):
    x1 = x1_ref[:]
    y1 = y1_ref[:]
    x2 = x2_ref[:]
    y2 = y2_ref[:]
    orig_sc = sc_ref[:]

    areas = (x2 - x1 + 1.0) * (y2 - y1 + 1.0)
    idx = (
        jax.lax.broadcasted_iota(jnp.int32, (_ROWS, 128), 0) * 128
        + jax.lax.broadcasted_iota(jnp.int32, (_ROWS, 128), 1)
    )
    lane = jax.lax.broadcasted_iota(jnp.int32, (1, 128), 1)
    sc0 = _allsum(jnp.where(idx == 0, orig_sc, 0.0))

    def step(i, live):
        mb = _allmax(live)
        eqm = live == mb
        selb = idx == _allmin(jnp.where(eqm, idx, _P))
        self32 = selb.astype(jnp.float32)
        bx1 = _allsum(x1 * self32)
        by1 = _allsum(y1 * self32)
        bx2 = _allsum(x2 * self32)
        by2 = _allsum(y2 * self32)
        bsc = jnp.where(mb > -1e29, mb, sc0)

        ix1 = jnp.maximum(bx1, x1)
        iy1 = jnp.maximum(by1, y1)
        ix2 = jnp.minimum(bx2, x2)
        iy2 = jnp.minimum(by2, y2)
        w = jnp.maximum(ix2 - ix1 + 1.0, 0.0)
        h = jnp.maximum(iy2 - iy1 + 1.0, 0.0)
        inter = w * h
        barea = (bx2 - bx1 + 1.0) * (by2 - by1 + 1.0)
        iou = inter / (barea + areas - inter)
        suppress = (iou > _IOU_THRESH) | selb
        live = jnp.where(suppress, _NEG, live)

        row = jnp.where(
            lane == 0,
            bsc[:1],
            jnp.where(
                lane == 1,
                bx1[:1],
                jnp.where(lane == 2, by1[:1], jnp.where(lane == 3, bx2[:1], by2[:1])),
            ),
        )
        out_ref[pl.ds(i, 1), :] = row
        return live

    jax.lax.fori_loop(0, _MAX_OUT, step, orig_sc)


def kernel(boxes, scores):
    pad = _P - _N
    x1 = jnp.pad(boxes[:, 0], (0, pad)).reshape(_ROWS, 128)
    y1 = jnp.pad(boxes[:, 1], (0, pad)).reshape(_ROWS, 128)
    x2 = jnp.pad(boxes[:, 2], (0, pad)).reshape(_ROWS, 128)
    y2 = jnp.pad(boxes[:, 3], (0, pad)).reshape(_ROWS, 128)
    sc = jnp.pad(scores, (0, pad), constant_values=_NEG).reshape(_ROWS, 128)

    out = pl.pallas_call(
        _nms_body,
        out_shape=jax.ShapeDtypeStruct((_MAX_OUT, 128), jnp.float32),
    )(x1, y1, x2, y2, sc)
    return out[:, :5]


# speedup vs baseline: 37.7888x; 1.5511x over previous
import jax
import jax.numpy as jnp
from jax.experimental import pallas as pl
from jax.experimental.pallas import tpu as pltpu

_N = 5000
_P = 5120
_ROWS = 40
_MAX_OUT = 300
_IOU_THRESH = 0.7
_NEG = -1e30


def _allmax(x):
    r = jnp.max(x.reshape(_ROWS // 8, 8, 128), axis=0)
    r = jnp.max(r, axis=0, keepdims=True)
    r = jnp.max(r, axis=1, keepdims=True)
    return jnp.broadcast_to(r.reshape(1, 1), x.shape)


def _allmin(x):
    r = jnp.min(x.reshape(_ROWS // 8, 8, 128), axis=0)
    r = jnp.min(r, axis=0, keepdims=True)
    r = jnp.min(r, axis=1, keepdims=True)
    return jnp.broadcast_to(r.reshape(1, 1), x.shape)


def _allsum(x):
    r = jnp.sum(x.reshape(_ROWS // 8, 8, 128), axis=0)
    r = jnp.sum(r, axis=0, keepdims=True)
    r = jnp.sum(r, axis=1, keepdims=True)
    return jnp.broadcast_to(r.reshape(1, 1), x.shape)


def _nms_body(bsm_ref, x1_ref, y1_ref, x2_ref, y2_ref, sc_ref, out_ref):
    x1 = x1_ref[:]
    y1 = y1_ref[:]
    x2 = x2_ref[:]
    y2 = y2_ref[:]
    orig_sc = sc_ref[:]

    areas = (x2 - x1 + 1.0) * (y2 - y1 + 1.0)
    idxf = (
        jax.lax.broadcasted_iota(jnp.int32, (_ROWS, 128), 0) * 128
        + jax.lax.broadcasted_iota(jnp.int32, (_ROWS, 128), 1)
    ).astype(jnp.float32)
    lane = jax.lax.broadcasted_iota(jnp.int32, (1, 128), 1)
    sc0 = _allsum(jnp.where(idxf == 0.0, orig_sc, 0.0))

    def step(i, live):
        mb = _allmax(live)
        eqm = live == mb
        pf = _allmin(jnp.where(eqm, idxf, float(_P)))
        p = pf[0, 0].astype(jnp.int32)
        bx1 = bsm_ref[0, p]
        by1 = bsm_ref[1, p]
        bx2 = bsm_ref[2, p]
        by2 = bsm_ref[3, p]

        ix1 = jnp.maximum(bx1, x1)
        iy1 = jnp.maximum(by1, y1)
        ix2 = jnp.minimum(bx2, x2)
        iy2 = jnp.minimum(by2, y2)
        w = jnp.maximum(ix2 - ix1 + 1.0, 0.0)
        h = jnp.maximum(iy2 - iy1 + 1.0, 0.0)
        inter = w * h
        barea = (bx2 - bx1 + 1.0) * (by2 - by1 + 1.0)
        iou = inter / (barea + areas - inter)
        suppress = (iou > _IOU_THRESH) | (idxf == pf)
        live = jnp.where(suppress, _NEG, live)

        bsc = jnp.where(mb > -1e29, mb, sc0)[0:1]
        row = jnp.where(
            lane == 0,
            bsc,
            jnp.where(
                lane == 1,
                bx1,
                jnp.where(lane == 2, by1, jnp.where(lane == 3, bx2, by2)),
            ),
        )
        out_ref[pl.ds(i, 1), :] = row
        return live

    jax.lax.fori_loop(0, _MAX_OUT, step, orig_sc)


def kernel(boxes, scores):
    pad = _P - _N
    boxes_pad = jnp.pad(boxes, ((0, pad), (0, 0)))
    x1 = boxes_pad[:, 0].reshape(_ROWS, 128)
    y1 = boxes_pad[:, 1].reshape(_ROWS, 128)
    x2 = boxes_pad[:, 2].reshape(_ROWS, 128)
    y2 = boxes_pad[:, 3].reshape(_ROWS, 128)
    sc = jnp.pad(scores, (0, pad), constant_values=_NEG).reshape(_ROWS, 128)

    out = pl.pallas_call(
        _nms_body,
        in_specs=[
            pl.BlockSpec(memory_space=pltpu.SMEM),
            pl.BlockSpec(memory_space=pltpu.VMEM),
            pl.BlockSpec(memory_space=pltpu.VMEM),
            pl.BlockSpec(memory_space=pltpu.VMEM),
            pl.BlockSpec(memory_space=pltpu.VMEM),
            pl.BlockSpec(memory_space=pltpu.VMEM),
        ],
        out_shape=jax.ShapeDtypeStruct((_MAX_OUT, 128), jnp.float32),
    )(boxes_pad.T, x1, y1, x2, y2, sc)
    return out[:, :5]


# speedup vs baseline: 39.7049x; 1.0507x over previous
import jax
import jax.numpy as jnp
from jax.experimental import pallas as pl
from jax.experimental.pallas import tpu as pltpu

_N = 5000
_P = 5120
_ROWS = 40
_MAX_OUT = 300
_IOU_THRESH = 0.7
_NEG = -1e30


def _allmax(x):
    r = jnp.max(x.reshape(_ROWS // 8, 8, 128), axis=0)
    r = jnp.max(r, axis=0, keepdims=True)
    r = jnp.max(r, axis=1, keepdims=True)
    return jnp.broadcast_to(r.reshape(1, 1), x.shape)


def _allmin(x):
    r = jnp.min(x.reshape(_ROWS // 8, 8, 128), axis=0)
    r = jnp.min(r, axis=0, keepdims=True)
    r = jnp.min(r, axis=1, keepdims=True)
    return jnp.broadcast_to(r.reshape(1, 1), x.shape)


def _allsum(x):
    r = jnp.sum(x.reshape(_ROWS // 8, 8, 128), axis=0)
    r = jnp.sum(r, axis=0, keepdims=True)
    r = jnp.sum(r, axis=1, keepdims=True)
    return jnp.broadcast_to(r.reshape(1, 1), x.shape)


def _nms_body(bsm_ref, x1_ref, y1_ref, x2_ref, y2_ref, sc_ref, out_ref):
    x1 = x1_ref[:]
    y1 = y1_ref[:]
    x2 = x2_ref[:]
    y2 = y2_ref[:]
    orig_sc = sc_ref[:]

    areas = (x2 - x1 + 1.0) * (y2 - y1 + 1.0)
    idxf = (
        jax.lax.broadcasted_iota(jnp.int32, (_ROWS, 128), 0) * 128
        + jax.lax.broadcasted_iota(jnp.int32, (_ROWS, 128), 1)
    ).astype(jnp.float32)
    rowf = jax.lax.broadcasted_iota(jnp.int32, (_ROWS, 128), 0).astype(jnp.float32)
    lane = jax.lax.broadcasted_iota(jnp.int32, (1, 128), 1)
    lanef = lane.astype(jnp.float32)
    sc0 = _allsum(jnp.where(idxf == 0.0, orig_sc, 0.0))

    def step(i, live):
        cm = jnp.max(live.reshape(_ROWS // 8, 8, 128), axis=0)
        cm = jnp.max(cm, axis=0, keepdims=True)
        m = jnp.max(cm, axis=1, keepdims=True)
        mb = jnp.broadcast_to(m.reshape(1, 1), live.shape)
        eqc = live == jnp.broadcast_to(cm, live.shape)
        car = jnp.min(
            jnp.where(eqc, rowf, float(_ROWS)).reshape(_ROWS // 8, 8, 128), axis=0
        )
        car = jnp.min(car, axis=0, keepdims=True)
        key = car * 128.0 + lanef
        pf = jnp.min(jnp.where(cm == mb[0:1], key, float(_P)), axis=1, keepdims=True)
        pfb = jnp.broadcast_to(pf.reshape(1, 1), live.shape)
        p = pfb.astype(jnp.int32)[0, 0]
        bx1 = bsm_ref[0, p]
        by1 = bsm_ref[1, p]
        bx2 = bsm_ref[2, p]
        by2 = bsm_ref[3, p]

        ix1 = jnp.maximum(bx1, x1)
        iy1 = jnp.maximum(by1, y1)
        ix2 = jnp.minimum(bx2, x2)
        iy2 = jnp.minimum(by2, y2)
        w = jnp.maximum(ix2 - ix1 + 1.0, 0.0)
        h = jnp.maximum(iy2 - iy1 + 1.0, 0.0)
        inter = w * h
        barea = (bx2 - bx1 + 1.0) * (by2 - by1 + 1.0)
        iou = inter / (barea + areas - inter)
        suppress = (iou > _IOU_THRESH) | (idxf == pfb)
        live = jnp.where(suppress, _NEG, live)

        bsc = jnp.where(mb > -1e29, mb, sc0)[0:1]
        row = jnp.where(
            lane == 0,
            bsc,
            jnp.where(
                lane == 1,
                bx1,
                jnp.where(lane == 2, by1, jnp.where(lane == 3, bx2, by2)),
            ),
        )
        out_ref[pl.ds(i, 1), :] = row
        return live

    jax.lax.fori_loop(0, _MAX_OUT, step, orig_sc)


def kernel(boxes, scores):
    pad = _P - _N
    boxes_pad = jnp.pad(boxes, ((0, pad), (0, 0)))
    x1 = boxes_pad[:, 0].reshape(_ROWS, 128)
    y1 = boxes_pad[:, 1].reshape(_ROWS, 128)
    x2 = boxes_pad[:, 2].reshape(_ROWS, 128)
    y2 = boxes_pad[:, 3].reshape(_ROWS, 128)
    sc = jnp.pad(scores, (0, pad), constant_values=_NEG).reshape(_ROWS, 128)

    out = pl.pallas_call(
        _nms_body,
        in_specs=[
            pl.BlockSpec(memory_space=pltpu.SMEM),
            pl.BlockSpec(memory_space=pltpu.VMEM),
            pl.BlockSpec(memory_space=pltpu.VMEM),
            pl.BlockSpec(memory_space=pltpu.VMEM),
            pl.BlockSpec(memory_space=pltpu.VMEM),
            pl.BlockSpec(memory_space=pltpu.VMEM),
        ],
        out_shape=jax.ShapeDtypeStruct((_MAX_OUT, 128), jnp.float32),
    )(boxes_pad.T, x1, y1, x2, y2, sc)
    return out[:, :5]
